# baseline (speedup 1.0000x reference)
import jax
import jax.numpy as jnp
from jax.experimental import pallas as pl
from jax.experimental.pallas import tpu as pltpu

N, C, H, W = 16, 512, 32, 32
ROWS = N * C
COLS = H * W
BR = 256


def _mlp_body(mask_ref, w1_ref, b1_ref, w2_ref, b2_ref, wfc_ref, bfc_ref,
              m_ref, s_ref):
    h = jax.nn.relu(
        jnp.dot(mask_ref[...], w1_ref[...], preferred_element_type=jnp.float32)
        + b1_ref[...])
    m = jax.nn.sigmoid(
        jnp.dot(h, w2_ref[...], preferred_element_type=jnp.float32)
        + b2_ref[...])
    s = jax.nn.sigmoid(
        jnp.dot(m, wfc_ref[...], preferred_element_type=jnp.float32)
        + bfc_ref[...])
    m_ref[...] = m
    s_ref[...] = s


def _exchange_body(sel_ref, lst_ref, gui_ref, out_lst_ref, out_gui_ref):
    cond = sel_ref[...] > 0.5
    l = lst_ref[...]
    g = gui_ref[...]
    out_lst_ref[...] = jnp.where(cond, g, l)
    out_gui_ref[...] = jnp.where(cond, l, g)


def kernel(lst, gui, mask, W1, b1, W2, b2, Wfc, bfc):
    m, s = pl.pallas_call(
        _mlp_body,
        out_shape=(
            jax.ShapeDtypeStruct((N, C), jnp.float32),
            jax.ShapeDtypeStruct((N, C), jnp.float32),
        ),
    )(mask, W1, b1.reshape(1, C), W2, b2.reshape(1, C),
      Wfc, bfc.reshape(1, C))

    BC = 128
    sel4 = s.reshape(N, C, 1, 1)
    out_lst, out_gui = pl.pallas_call(
        _exchange_body,
        grid=(N, C // BC),
        in_specs=[
            pl.BlockSpec((1, BC, 1, 1), lambda n, c: (n, c, 0, 0)),
            pl.BlockSpec((1, BC, H, W), lambda n, c: (n, c, 0, 0)),
            pl.BlockSpec((1, BC, H, W), lambda n, c: (n, c, 0, 0)),
        ],
        out_specs=[
            pl.BlockSpec((1, BC, H, W), lambda n, c: (n, c, 0, 0)),
            pl.BlockSpec((1, BC, H, W), lambda n, c: (n, c, 0, 0)),
        ],
        out_shape=(
            jax.ShapeDtypeStruct((N, C, H, W), jnp.float32),
            jax.ShapeDtypeStruct((N, C, H, W), jnp.float32),
        ),
    )(sel4, lst, gui)

    return (out_lst, out_gui, m)


# speedup vs baseline: 9.3148x; 9.3148x over previous
import jax
import jax.numpy as jnp
from jax.experimental import pallas as pl
from jax.experimental.pallas import tpu as pltpu

N, C, H, W = 16, 512, 32, 32
BH = 16


def _mlp_body(mask_ref, w1_ref, b1_ref, w2_ref, b2_ref, wfc_ref, bfc_ref,
              m_ref, s_ref):
    h = jax.nn.relu(
        jnp.dot(mask_ref[...], w1_ref[...], preferred_element_type=jnp.float32)
        + b1_ref[...])
    m = jax.nn.sigmoid(
        jnp.dot(h, w2_ref[...], preferred_element_type=jnp.float32)
        + b2_ref[...])
    s = jax.nn.sigmoid(
        jnp.dot(m, wfc_ref[...], preferred_element_type=jnp.float32)
        + bfc_ref[...])
    m_ref[...] = m
    s_ref[...] = s


def _exchange_body(sel_ref, lst_ref, gui_ref, out_lst_ref, out_gui_ref):
    n = pl.program_id(0)
    cond = (sel_ref[n, :] > 0.5)[None, None, None, :]
    l = lst_ref[...]
    g = gui_ref[...]
    out_lst_ref[...] = jnp.where(cond, g, l)
    out_gui_ref[...] = jnp.where(cond, l, g)


def kernel(lst, gui, mask, W1, b1, W2, b2, Wfc, bfc):
    m, s = pl.pallas_call(
        _mlp_body,
        out_shape=(
            jax.ShapeDtypeStruct((N, C), jnp.float32),
            jax.ShapeDtypeStruct((N, C), jnp.float32),
        ),
    )(mask, W1, b1.reshape(1, C), W2, b2.reshape(1, C),
      Wfc, bfc.reshape(1, C))

    lst_t = lst.transpose(0, 2, 3, 1)
    gui_t = gui.transpose(0, 2, 3, 1)

    out_lst_t, out_gui_t = pl.pallas_call(
        _exchange_body,
        grid=(N, H // BH),
        in_specs=[
            pl.BlockSpec((N, C), lambda n, h: (0, 0)),
            pl.BlockSpec((1, BH, W, C), lambda n, h: (n, h, 0, 0)),
            pl.BlockSpec((1, BH, W, C), lambda n, h: (n, h, 0, 0)),
        ],
        out_specs=[
            pl.BlockSpec((1, BH, W, C), lambda n, h: (n, h, 0, 0)),
            pl.BlockSpec((1, BH, W, C), lambda n, h: (n, h, 0, 0)),
        ],
        out_shape=(
            jax.ShapeDtypeStruct((N, H, W, C), jnp.float32),
            jax.ShapeDtypeStruct((N, H, W, C), jnp.float32),
        ),
    )(s, lst_t, gui_t)

    return (out_lst_t.transpose(0, 3, 1, 2),
            out_gui_t.transpose(0, 3, 1, 2), m)


# speedup vs baseline: 10.6202x; 1.1401x over previous
import jax
import jax.numpy as jnp
from jax.experimental import pallas as pl
from jax.experimental.pallas import tpu as pltpu

N, C, H, W = 16, 512, 32, 32
BH = 16


def _mlp_body(mask_ref, w1_ref, b1_ref, w2_ref, b2_ref, wfc_ref, bfc_ref,
              m_ref, s_ref):
    h = jax.nn.relu(
        jnp.dot(mask_ref[...], w1_ref[...], preferred_element_type=jnp.float32)
        + b1_ref[...])
    m = jax.nn.sigmoid(
        jnp.dot(h, w2_ref[...], preferred_element_type=jnp.float32)
        + b2_ref[...])
    s = jax.nn.sigmoid(
        jnp.dot(m, wfc_ref[...], preferred_element_type=jnp.float32)
        + bfc_ref[...])
    m_ref[...] = m
    s_ref[...] = s


def _exchange_body(sel_ref, lst_ref, gui_ref, out_lst_ref, out_gui_ref):
    n = pl.program_id(0)
    cond = (sel_ref[n, :] > 0.5)[None, None, None, :]
    l = lst_ref[...]
    g = gui_ref[...]
    out_lst_ref[...] = jnp.where(cond, g, l)
    out_gui_ref[...] = jnp.where(cond, l, g)


def _exchange_call(s, lst_t, gui_t):
    return pl.pallas_call(
        _exchange_body,
        grid=(N,),
        in_specs=[
            pl.BlockSpec((N, C), lambda n: (0, 0)),
            pl.BlockSpec((1, H, W, C), lambda n: (n, 0, 0, 0)),
            pl.BlockSpec((1, H, W, C), lambda n: (n, 0, 0, 0)),
        ],
        out_specs=[
            pl.BlockSpec((1, H, W, C), lambda n: (n, 0, 0, 0)),
            pl.BlockSpec((1, H, W, C), lambda n: (n, 0, 0, 0)),
        ],
        out_shape=(
            jax.ShapeDtypeStruct((N, H, W, C), jnp.float32),
            jax.ShapeDtypeStruct((N, H, W, C), jnp.float32),
        ),
    )(s, lst_t, gui_t)


def kernel(lst, gui, mask, W1, b1, W2, b2, Wfc, bfc):
    m, s = pl.pallas_call(
        _mlp_body,
        out_shape=(
            jax.ShapeDtypeStruct((N, C), jnp.float32),
            jax.ShapeDtypeStruct((N, C), jnp.float32),
        ),
    )(mask, W1, b1.reshape(1, C), W2, b2.reshape(1, C),
      Wfc, bfc.reshape(1, C))

    lst_t = lst.transpose(0, 2, 3, 1)
    gui_t = gui.transpose(0, 2, 3, 1)

    out_lst_t, out_gui_t = _exchange_call(s, lst_t, gui_t)

    return (out_lst_t.transpose(0, 3, 1, 2),
            out_gui_t.transpose(0, 3, 1, 2), m)


# speedup vs baseline: 10.9277x; 1.0290x over previous
import jax
import jax.numpy as jnp
from jax.experimental import pallas as pl
from jax.experimental.pallas import tpu as pltpu

N, C, H, W = 16, 512, 32, 32


def _fused_body(mask_ref, w1_ref, b1_ref, w2_ref, b2_ref, wfc_ref, bfc_ref,
                lst_ref, gui_ref, m_ref, out_lst_ref, out_gui_ref, sel_ref):
    n = pl.program_id(0)

    @pl.when(n == 0)
    def _mlp():
        h = jax.nn.relu(
            jnp.dot(mask_ref[...], w1_ref[...],
                    preferred_element_type=jnp.float32) + b1_ref[...])
        m = jax.nn.sigmoid(
            jnp.dot(h, w2_ref[...],
                    preferred_element_type=jnp.float32) + b2_ref[...])
        s = jax.nn.sigmoid(
            jnp.dot(m, wfc_ref[...],
                    preferred_element_type=jnp.float32) + bfc_ref[...])
        m_ref[...] = m
        sel_ref[...] = s

    cond = (sel_ref[n, :] > 0.5)[None, None, None, :]
    l = lst_ref[...]
    g = gui_ref[...]
    out_lst_ref[...] = jnp.where(cond, g, l)
    out_gui_ref[...] = jnp.where(cond, l, g)


def kernel(lst, gui, mask, W1, b1, W2, b2, Wfc, bfc):
    lst_t = lst.transpose(0, 2, 3, 1)
    gui_t = gui.transpose(0, 2, 3, 1)

    m, out_lst_t, out_gui_t = pl.pallas_call(
        _fused_body,
        grid=(N,),
        in_specs=[
            pl.BlockSpec((N, 1024), lambda n: (0, 0)),
            pl.BlockSpec((1024, C), lambda n: (0, 0)),
            pl.BlockSpec((1, C), lambda n: (0, 0)),
            pl.BlockSpec((C, C), lambda n: (0, 0)),
            pl.BlockSpec((1, C), lambda n: (0, 0)),
            pl.BlockSpec((C, C), lambda n: (0, 0)),
            pl.BlockSpec((1, C), lambda n: (0, 0)),
            pl.BlockSpec((1, H, W, C), lambda n: (n, 0, 0, 0)),
            pl.BlockSpec((1, H, W, C), lambda n: (n, 0, 0, 0)),
        ],
        out_specs=[
            pl.BlockSpec((N, C), lambda n: (0, 0)),
            pl.BlockSpec((1, H, W, C), lambda n: (n, 0, 0, 0)),
            pl.BlockSpec((1, H, W, C), lambda n: (n, 0, 0, 0)),
        ],
        out_shape=(
            jax.ShapeDtypeStruct((N, C), jnp.float32),
            jax.ShapeDtypeStruct((N, H, W, C), jnp.float32),
            jax.ShapeDtypeStruct((N, H, W, C), jnp.float32),
        ),
        scratch_shapes=[pltpu.VMEM((N, C), jnp.float32)],
    )(mask, W1, b1.reshape(1, C), W2, b2.reshape(1, C),
      Wfc, bfc.reshape(1, C), lst_t, gui_t)

    return (out_lst_t.transpose(0, 3, 1, 2),
            out_gui_t.transpose(0, 3, 1, 2), m)


# speedup vs baseline: 12.3584x; 1.1309x over previous
import jax
import jax.numpy as jnp
from jax.experimental import pallas as pl
from jax.experimental.pallas import tpu as pltpu

N, C, H, W = 16, 512, 32, 32
BN = 2


def _fused_body(mask_ref, w1_ref, b1_ref, w2_ref, b2_ref, wfc_ref, bfc_ref,
                lst_ref, gui_ref, m_ref, out_lst_ref, out_gui_ref, sel_ref):
    n = pl.program_id(0)

    @pl.when(n == 0)
    def _mlp():
        h = jax.nn.relu(
            jnp.dot(mask_ref[...], w1_ref[...],
                    preferred_element_type=jnp.float32) + b1_ref[...])
        m = jax.nn.sigmoid(
            jnp.dot(h, w2_ref[...],
                    preferred_element_type=jnp.float32) + b2_ref[...])
        s = jax.nn.sigmoid(
            jnp.dot(m, wfc_ref[...],
                    preferred_element_type=jnp.float32) + bfc_ref[...])
        m_ref[...] = m
        sel_ref[...] = s

    rows = [sel_ref[n * BN + j, :][None, :] for j in range(BN)]
    cond = (jnp.concatenate(rows, axis=0) > 0.5)[:, None, None, :]
    l = lst_ref[...]
    g = gui_ref[...]
    out_lst_ref[...] = jnp.where(cond, g, l)
    out_gui_ref[...] = jnp.where(cond, l, g)


def kernel(lst, gui, mask, W1, b1, W2, b2, Wfc, bfc):
    lst_t = lst.transpose(0, 2, 3, 1)
    gui_t = gui.transpose(0, 2, 3, 1)

    m, out_lst_t, out_gui_t = pl.pallas_call(
        _fused_body,
        grid=(N // BN,),
        in_specs=[
            pl.BlockSpec((N, 1024), lambda n: (0, 0)),
            pl.BlockSpec((1024, C), lambda n: (0, 0)),
            pl.BlockSpec((1, C), lambda n: (0, 0)),
            pl.BlockSpec((C, C), lambda n: (0, 0)),
            pl.BlockSpec((1, C), lambda n: (0, 0)),
            pl.BlockSpec((C, C), lambda n: (0, 0)),
            pl.BlockSpec((1, C), lambda n: (0, 0)),
            pl.BlockSpec((BN, H, W, C), lambda n: (n, 0, 0, 0)),
            pl.BlockSpec((BN, H, W, C), lambda n: (n, 0, 0, 0)),
        ],
        out_specs=[
            pl.BlockSpec((N, C), lambda n: (0, 0)),
            pl.BlockSpec((BN, H, W, C), lambda n: (n, 0, 0, 0)),
            pl.BlockSpec((BN, H, W, C), lambda n: (n, 0, 0, 0)),
        ],
        out_shape=(
            jax.ShapeDtypeStruct((N, C), jnp.float32),
            jax.ShapeDtypeStruct((N, H, W, C), jnp.float32),
            jax.ShapeDtypeStruct((N, H, W, C), jnp.float32),
        ),
        scratch_shapes=[pltpu.VMEM((N, C), jnp.float32)],
    )(mask, W1, b1.reshape(1, C), W2, b2.reshape(1, C),
      Wfc, bfc.reshape(1, C), lst_t, gui_t)

    return (out_lst_t.transpose(0, 3, 1, 2),
            out_gui_t.transpose(0, 3, 1, 2), m)
